# Initial kernel scaffold; baseline (speedup 1.0000x reference)
import functools

import jax
import jax.numpy as jnp
from jax import lax
from jax.experimental import pallas as pl
from jax.experimental.pallas import tpu as pltpu
from jax.experimental.pallas import tpu_sc as plsc

CAT = 26
NUMD = 50
VOCAB = 100000
EMB = 16
FDIM = 128
CAT_EMB = CAT * EMB


@functools.lru_cache(maxsize=None)
def _make_gather(n_rows: int):
    info = plsc.get_sparse_core_info()
    nw = info.num_cores * info.num_subcores
    rows_per_w = n_rows // nw
    assert rows_per_w * nw == n_rows
    chunk = 3328
    n_chunks = rows_per_w // chunk
    assert n_chunks * chunk == rows_per_w
    mesh = plsc.VectorSubcoreMesh(core_axis_name="c", subcore_axis_name="s")

    @functools.partial(
        pl.kernel,
        mesh=mesh,
        out_type=jax.ShapeDtypeStruct((n_rows, EMB), jnp.float32),
        scratch_types=[
            pltpu.VMEM((chunk,), jnp.int32),
            pltpu.VMEM((chunk, EMB), jnp.float32),
            pltpu.SemaphoreType.DMA,
        ],
    )
    def gather_k(idx_hbm, table_hbm, out_hbm, idx_v, rows_v, sem):
        wid = lax.axis_index("s") * info.num_cores + lax.axis_index("c")
        w_base = wid * rows_per_w

        def body(c, carry):
            base = w_base + c * chunk
            pltpu.sync_copy(idx_hbm.at[pl.ds(base, chunk)], idx_v)
            pltpu.async_copy(table_hbm.at[idx_v], rows_v, sem).wait()
            pltpu.sync_copy(rows_v, out_hbm.at[pl.ds(base, chunk)])
            return carry

        lax.fori_loop(0, n_chunks, body, 0)

    return gather_k


_TM = 2048


def _matmul(cat_emb, num_x, Wn, bn, Wocat, Wonum, bo):
    n = cat_emb.shape[0]

    def body(cat_ref, num_ref, wn_ref, bn_ref, wc_ref, wo_ref, bo_ref, out_ref):
        num_emb = (
            jnp.dot(num_ref[...], wn_ref[...], preferred_element_type=jnp.float32)
            + bn_ref[...]
        )
        acc = jnp.dot(cat_ref[...], wc_ref[...], preferred_element_type=jnp.float32)
        acc = acc + jnp.dot(num_emb, wo_ref[...], preferred_element_type=jnp.float32)
        out_ref[...] = acc + bo_ref[...]

    return pl.pallas_call(
        body,
        grid=(n // _TM,),
        in_specs=[
            pl.BlockSpec((_TM, CAT_EMB), lambda i: (i, 0)),
            pl.BlockSpec((_TM, NUMD), lambda i: (i, 0)),
            pl.BlockSpec((NUMD, NUMD), lambda i: (0, 0)),
            pl.BlockSpec((1, NUMD), lambda i: (0, 0)),
            pl.BlockSpec((CAT_EMB, FDIM), lambda i: (0, 0)),
            pl.BlockSpec((NUMD, FDIM), lambda i: (0, 0)),
            pl.BlockSpec((1, FDIM), lambda i: (0, 0)),
        ],
        out_specs=pl.BlockSpec((_TM, FDIM), lambda i: (i, 0)),
        out_shape=jax.ShapeDtypeStruct((n, FDIM), jnp.float32),
    )(cat_emb, num_x, Wn, bn, Wocat, Wonum, bo)


def kernel(x, tables, Wn, bn, Wo, bo):
    b, s, _ = x.shape
    n = b * s
    xf = x.reshape(n, CAT + NUMD)
    idx = (
        xf[:, :CAT].astype(jnp.int32)
        + jnp.arange(CAT, dtype=jnp.int32) * VOCAB
    ).reshape(-1)
    table_flat = tables.reshape(CAT * VOCAB, EMB)
    rows = _make_gather(n * CAT)(idx, table_flat)
    cat_emb = rows.reshape(n, CAT_EMB)
    out = _matmul(
        cat_emb,
        xf[:, CAT:],
        Wn,
        bn.reshape(1, NUMD),
        Wo[:CAT_EMB],
        Wo[CAT_EMB:],
        bo.reshape(1, FDIM),
    )
    return out.reshape(b, s, FDIM)


# speedup vs baseline: 8.2570x; 8.2570x over previous
import functools

import jax
import jax.numpy as jnp
from jax import lax
from jax.experimental import pallas as pl
from jax.experimental.pallas import tpu as pltpu
from jax.experimental.pallas import tpu_sc as plsc

CAT = 26
NUMD = 50
VOCAB = 100000
EMB = 16
FDIM = 128
CAT_EMB = CAT * EMB


@functools.lru_cache(maxsize=None)
def _make_gather(n_rows: int):
    info = plsc.get_sparse_core_info()
    nw = info.num_cores * info.num_subcores
    rows_per_w = n_rows // nw
    assert rows_per_w * nw == n_rows
    chunk = 3328
    n_chunks = rows_per_w // chunk
    assert n_chunks * chunk == rows_per_w
    mesh = plsc.VectorSubcoreMesh(core_axis_name="c", subcore_axis_name="s")

    @functools.partial(
        pl.kernel,
        mesh=mesh,
        compiler_params=pltpu.CompilerParams(use_tc_tiling_on_sc=False),
        out_type=jax.ShapeDtypeStruct((n_rows, EMB), jnp.float32),
        scratch_types=[
            pltpu.VMEM((chunk,), jnp.int32),
            pltpu.VMEM((chunk, EMB), jnp.float32),
            pltpu.SemaphoreType.DMA,
        ],
    )
    def gather_k(idx_hbm, table_hbm, out_hbm, idx_v, rows_v, sem):
        wid = lax.axis_index("s") * info.num_cores + lax.axis_index("c")
        w_base = wid * rows_per_w

        def body(c, carry):
            base = w_base + c * chunk
            pltpu.sync_copy(idx_hbm.at[pl.ds(base, chunk)], idx_v)
            pltpu.async_copy(table_hbm.at[idx_v], rows_v, sem).wait()
            pltpu.sync_copy(rows_v, out_hbm.at[pl.ds(base, chunk)])
            return carry

        lax.fori_loop(0, n_chunks, body, 0)

    return gather_k


_TM = 2048


def _matmul(cat_emb, num_x, Wn, bn, Wocat, Wonum, bo):
    n = cat_emb.shape[0]

    def body(cat_ref, num_ref, wn_ref, bn_ref, wc_ref, wo_ref, bo_ref, out_ref):
        num_emb = (
            jnp.dot(num_ref[...], wn_ref[...], preferred_element_type=jnp.float32)
            + bn_ref[...]
        )
        acc = jnp.dot(cat_ref[...], wc_ref[...], preferred_element_type=jnp.float32)
        acc = acc + jnp.dot(num_emb, wo_ref[...], preferred_element_type=jnp.float32)
        out_ref[...] = acc + bo_ref[...]

    return pl.pallas_call(
        body,
        grid=(n // _TM,),
        in_specs=[
            pl.BlockSpec((_TM, CAT_EMB), lambda i: (i, 0)),
            pl.BlockSpec((_TM, NUMD), lambda i: (i, 0)),
            pl.BlockSpec((NUMD, NUMD), lambda i: (0, 0)),
            pl.BlockSpec((1, NUMD), lambda i: (0, 0)),
            pl.BlockSpec((CAT_EMB, FDIM), lambda i: (0, 0)),
            pl.BlockSpec((NUMD, FDIM), lambda i: (0, 0)),
            pl.BlockSpec((1, FDIM), lambda i: (0, 0)),
        ],
        out_specs=pl.BlockSpec((_TM, FDIM), lambda i: (i, 0)),
        out_shape=jax.ShapeDtypeStruct((n, FDIM), jnp.float32),
    )(cat_emb, num_x, Wn, bn, Wocat, Wonum, bo)


def kernel(x, tables, Wn, bn, Wo, bo):
    b, s, _ = x.shape
    n = b * s
    xf = x.reshape(n, CAT + NUMD)
    idx = (
        xf[:, :CAT].astype(jnp.int32)
        + jnp.arange(CAT, dtype=jnp.int32) * VOCAB
    ).reshape(-1)
    table_flat = tables.reshape(CAT * VOCAB, EMB)
    rows = _make_gather(n * CAT)(idx, table_flat)
    cat_emb = rows.reshape(n, CAT_EMB)
    out = _matmul(
        cat_emb,
        xf[:, CAT:],
        Wn,
        bn.reshape(1, NUMD),
        Wo[:CAT_EMB],
        Wo[CAT_EMB:],
        bo.reshape(1, FDIM),
    )
    return out.reshape(b, s, FDIM)
